# Initial kernel scaffold; baseline (speedup 1.0000x reference)
import jax
import jax.numpy as jnp
from jax.experimental import pallas as pl


def kernel(x, params):
    raise NotImplementedError("write your pallas kernel here")


# speedup vs baseline: 10.7639x; 10.7639x over previous
import functools

import jax
import jax.numpy as jnp
from jax.experimental import pallas as pl


def _knn_kernel(cq_ref, ckT_ref, idx_ref, *, n, q, k):
    cq = cq_ref[...]
    ckT = ckT_ref[...]
    sq = (cq[:, 0:1] * cq[:, 0:1] + cq[:, 1:2] * cq[:, 1:2]
          + cq[:, 2:3] * cq[:, 2:3])
    sk = (ckT[0:1, :] * ckT[0:1, :] + ckT[1:2, :] * ckT[1:2, :]
          + ckT[2:3, :] * ckT[2:3, :])
    cqb = cq.astype(jnp.bfloat16).astype(jnp.float32)
    ckb = ckT.astype(jnp.bfloat16).astype(jnp.float32)
    dot = (cqb[:, 0:1] * ckb[0:1, :]
           + cqb[:, 1:2] * ckb[1:2, :]
           + cqb[:, 2:3] * ckb[2:3, :])
    d = sq + sk - 2.0 * dot
    iota = jax.lax.broadcasted_iota(jnp.int32, (q, n), 1)
    lane8 = jax.lax.broadcasted_iota(jnp.int32, (q, 8), 1)
    out = jnp.zeros((q, 8), jnp.int32)
    for j in range(k):
        m = jnp.max(d, axis=1, keepdims=True)
        cand = jnp.where(d == m, iota, n)
        am = jnp.min(cand, axis=1, keepdims=True)
        out = jnp.where(lane8 == j, am, out)
        d = jnp.where(iota == am, -jnp.inf, d)
    idx_ref[...] = out


def _knn(coords, k=4, interpret=False):
    n = coords.shape[0]
    q = min(n, 256)
    ckT = coords.T
    idx8 = pl.pallas_call(
        functools.partial(_knn_kernel, n=n, q=q, k=k),
        grid=(n // q,),
        in_specs=[
            pl.BlockSpec((q, 3), lambda i: (i, 0)),
            pl.BlockSpec((3, n), lambda i: (0, 0)),
        ],
        out_specs=pl.BlockSpec((q, 8), lambda i: (i, 0)),
        out_shape=jax.ShapeDtypeStruct((n, 8), jnp.int32),
        interpret=interpret,
    )(coords, ckT)
    return idx8[:, :k]


def _lrelu(x, s):
    return jnp.where(x >= 0, x, s * x)


def _bn(y, gamma, beta):
    m = y.mean(axis=(0, 2, 3), keepdims=True)
    v = y.var(axis=(0, 2, 3), keepdims=True)
    y = (y - m) / jnp.sqrt(v + 1e-6)
    return y * gamma[None, :, None, None] + beta[None, :, None, None]


def _smlp(p, x, act_slope=None):
    y = jnp.einsum('oi,bink->bonk', p['W'], x) + p['b'][None, :, None, None]
    y = _bn(y, p['gamma'], p['beta'])
    if act_slope is not None:
        y = _lrelu(y, act_slope)
    return y


def _gather(vals, idx):
    g = jax.vmap(lambda v, i: v[i])(vals, idx)
    return jnp.transpose(g, (0, 3, 1, 2))


def _attentive_pool(p_score, p_mlp, x):
    xp = jnp.transpose(x, (0, 2, 3, 1))
    sc = xp @ p_score['W'].T + p_score['b']
    sc = jax.nn.softmax(sc, axis=-2)
    sc = jnp.transpose(sc, (0, 3, 1, 2))
    feat = jnp.sum(sc * x, axis=-1, keepdims=True)
    return _smlp(p_mlp, feat, act_slope=0.2)


def _lfa(p, coords, features, k, interpret=False):
    idx = _knn(coords[0], k, interpret=interpret)[None]
    x = _smlp(p['mlp1'], features, act_slope=0.2)
    nb_coords = _gather(coords, idx)
    ext_coords = jnp.broadcast_to(
        jnp.transpose(coords, (0, 2, 1))[..., None], nb_coords.shape)
    rel_pos = ext_coords - nb_coords
    rel_dist = jnp.sqrt(jnp.maximum(
        jnp.sum(rel_pos * rel_pos, axis=1, keepdims=True), 1e-12))
    rel_feat = jnp.concatenate([rel_dist, rel_pos, ext_coords, nb_coords], axis=1)
    rel_feat = _smlp(p['lse1_mlp'], rel_feat, act_slope=0.2)
    nb_feat = _gather(jnp.transpose(x[..., 0], (0, 2, 1)), idx)
    x = _attentive_pool(p['pool1_score'], p['pool1_mlp'],
                        jnp.concatenate([nb_feat, rel_feat], axis=1))
    rel_feat2 = _smlp(p['lse2_mlp'], rel_feat, act_slope=0.2)
    nb_feat2 = _gather(jnp.transpose(x[..., 0], (0, 2, 1)), idx)
    x = _attentive_pool(p['pool2_score'], p['pool2_mlp'],
                        jnp.concatenate([nb_feat2, rel_feat2], axis=1))
    return _lrelu(_smlp(p['mlp2'], x) + _smlp(p['shortcut'], features), 0.01)


def _run(x, params, interpret=False):
    coords = x[..., :3]
    h = x @ params['fc0']['W'].T + params['fc0']['b']
    h = jnp.transpose(h, (0, 2, 1))[..., None]
    h = _lrelu(_bn(h, params['bn0']['gamma'], params['bn0']['beta']), 0.2)
    N = x.shape[1]
    ratio = 1
    for name in ('lfa0', 'lfa1', 'lfa2', 'lfa3'):
        n = N // ratio
        h = _lfa(params[name], coords[:, :n], h, 4, interpret=interpret)
        ratio *= 4
        h = h[:, :, :N // ratio]
    return _smlp(params['mlp'], h, act_slope=0.2)


def kernel(x, params):
    return _run(x, params)


# speedup vs baseline: 18.6332x; 1.7311x over previous
import functools

import jax
import jax.numpy as jnp
from jax.experimental import pallas as pl

_EPS = 1e-6
_HI = jax.lax.Precision.HIGHEST


def _knn_kernel(cq_ref, ckT_ref, idx_ref, *, n, q, k):
    cq = cq_ref[...]
    ckT = ckT_ref[...]
    sq = (cq[:, 0:1] * cq[:, 0:1] + cq[:, 1:2] * cq[:, 1:2]
          + cq[:, 2:3] * cq[:, 2:3])
    sk = (ckT[0:1, :] * ckT[0:1, :] + ckT[1:2, :] * ckT[1:2, :]
          + ckT[2:3, :] * ckT[2:3, :])
    cqb = cq.astype(jnp.bfloat16).astype(jnp.float32)
    ckb = ckT.astype(jnp.bfloat16).astype(jnp.float32)
    dot = (cqb[:, 0:1] * ckb[0:1, :]
           + cqb[:, 1:2] * ckb[1:2, :]
           + cqb[:, 2:3] * ckb[2:3, :])
    d = sq + sk - 2.0 * dot
    iota = jax.lax.broadcasted_iota(jnp.int32, (q, n), 1)
    lane8 = jax.lax.broadcasted_iota(jnp.int32, (q, 8), 1)
    out = jnp.zeros((q, 8), jnp.int32)
    for j in range(k):
        m = jnp.max(d, axis=1, keepdims=True)
        cand = jnp.where(d == m, iota, n)
        am = jnp.min(cand, axis=1, keepdims=True)
        out = jnp.where(lane8 == j, am, out)
        d = jnp.where(iota == am, -jnp.inf, d)
    idx_ref[...] = out


def _knn(coords, ckT, k=4, interpret=False):
    n = coords.shape[0]
    q = min(n, 256)
    idx8 = pl.pallas_call(
        functools.partial(_knn_kernel, n=n, q=q, k=k),
        grid=(n // q,),
        in_specs=[
            pl.BlockSpec((q, 3), lambda i: (i, 0)),
            pl.BlockSpec((3, n), lambda i: (0, 0)),
        ],
        out_specs=pl.BlockSpec((q, 8), lambda i: (i, 0)),
        out_shape=jax.ShapeDtypeStruct((n, 8), jnp.int32),
        interpret=interpret,
    )(coords, ckT)
    return idx8


def _lrelu(y, s):
    return jnp.where(y >= 0, y, s * y)


def _bn_apply(ys, gamma, beta):
    cnt = sum(y.shape[1] for y in ys)
    m = sum(jnp.sum(y, axis=1, keepdims=True) for y in ys) / cnt
    v = sum(jnp.sum((y - m) ** 2, axis=1, keepdims=True) for y in ys) / cnt
    inv = 1.0 / jnp.sqrt(v + _EPS)
    return [(y - m) * inv * gamma + beta for y in ys]


def _linear(w_ref, b_ref, x):
    return jnp.dot(w_ref[...], x, precision=_HI) + b_ref[...]


def _smlp_kernel(f_ref, w_ref, b_ref, g_ref, be_ref, o_ref, *, slope):
    y = _linear(w_ref, b_ref, f_ref[...])
    (y,) = _bn_apply([y], g_ref[...], be_ref[...])
    o_ref[...] = _lrelu(y, slope)


def _smlp_call(p, f, slope=0.2, interpret=False):
    cout = p['W'].shape[0]
    n = f.shape[1]
    return pl.pallas_call(
        functools.partial(_smlp_kernel, slope=slope),
        out_shape=jax.ShapeDtypeStruct((cout, n), jnp.float32),
        interpret=interpret,
    )(f, p['W'], p['b'].reshape(-1, 1), p['gamma'].reshape(-1, 1),
      p['beta'].reshape(-1, 1))


def _mid_kernel(ckT_ref, nbc_ref, nf1_ref,
                wl1_ref, bl1_ref, gl1_ref, bel1_ref,
                ws1_ref, bs1_ref,
                wp1_ref, bp1_ref, gp1_ref, bep1_ref,
                wl2_ref, bl2_ref, gl2_ref, bel2_ref,
                x2_ref, r2_ref, *, h):
    ckT = ckT_ref[...]
    y1 = []
    for j in range(4):
        nbc = nbc_ref[j]
        rp = ckT - nbc
        rd = jnp.sqrt(jnp.maximum(
            jnp.sum(rp * rp, axis=0, keepdims=True), 1e-12))
        rel = jnp.concatenate([rd, rp, ckT, nbc], axis=0)
        y1.append(_linear(wl1_ref, bl1_ref, rel))
    r1 = [_lrelu(y, 0.2)
          for y in _bn_apply(y1, gl1_ref[...], bel1_ref[...])]
    p1 = [jnp.concatenate([nf1_ref[j], r1[j]], axis=0) for j in range(4)]
    sc = [_linear(ws1_ref, bs1_ref, p) for p in p1]
    mx = jnp.maximum(jnp.maximum(sc[0], sc[1]), jnp.maximum(sc[2], sc[3]))
    e = [jnp.exp(s - mx) for s in sc]
    z = e[0] + e[1] + e[2] + e[3]
    feat = sum(ei / z * pi for ei, pi in zip(e, p1))
    y2 = _linear(wp1_ref, bp1_ref, feat)
    (y2,) = _bn_apply([y2], gp1_ref[...], bep1_ref[...])
    x2_ref[...] = _lrelu(y2, 0.2)
    y3 = [_linear(wl2_ref, bl2_ref, r) for r in r1]
    r2 = [_lrelu(y, 0.2)
          for y in _bn_apply(y3, gl2_ref[...], bel2_ref[...])]
    for j in range(4):
        r2_ref[j] = r2[j]


def _mid_call(p, ckT, nbc, nf1, interpret=False):
    h = p['lse1_mlp']['W'].shape[0]
    n = ckT.shape[1]
    rs = lambda a: a.reshape(-1, 1)
    return pl.pallas_call(
        functools.partial(_mid_kernel, h=h),
        out_shape=(jax.ShapeDtypeStruct((h, n), jnp.float32),
                   jax.ShapeDtypeStruct((4, h, n), jnp.float32)),
        interpret=interpret,
    )(ckT, nbc, nf1,
      p['lse1_mlp']['W'], rs(p['lse1_mlp']['b']),
      rs(p['lse1_mlp']['gamma']), rs(p['lse1_mlp']['beta']),
      p['pool1_score']['W'], rs(p['pool1_score']['b']),
      p['pool1_mlp']['W'], rs(p['pool1_mlp']['b']),
      rs(p['pool1_mlp']['gamma']), rs(p['pool1_mlp']['beta']),
      p['lse2_mlp']['W'], rs(p['lse2_mlp']['b']),
      rs(p['lse2_mlp']['gamma']), rs(p['lse2_mlp']['beta']))


def _out_kernel(nf2_ref, r2_ref, f_ref,
                ws2_ref, bs2_ref,
                wp2_ref, bp2_ref, gp2_ref, bep2_ref,
                wm2_ref, bm2_ref, gm2_ref, bem2_ref,
                wsh_ref, bsh_ref, gsh_ref, besh_ref,
                o_ref):
    p2 = [jnp.concatenate([nf2_ref[j], r2_ref[j]], axis=0) for j in range(4)]
    sc = [_linear(ws2_ref, bs2_ref, p) for p in p2]
    mx = jnp.maximum(jnp.maximum(sc[0], sc[1]), jnp.maximum(sc[2], sc[3]))
    e = [jnp.exp(s - mx) for s in sc]
    z = e[0] + e[1] + e[2] + e[3]
    feat = sum(ei / z * pi for ei, pi in zip(e, p2))
    y = _linear(wp2_ref, bp2_ref, feat)
    (y,) = _bn_apply([y], gp2_ref[...], bep2_ref[...])
    x3 = _lrelu(y, 0.2)
    m2 = _linear(wm2_ref, bm2_ref, x3)
    (m2,) = _bn_apply([m2], gm2_ref[...], bem2_ref[...])
    sh = _linear(wsh_ref, bsh_ref, f_ref[...])
    (sh,) = _bn_apply([sh], gsh_ref[...], besh_ref[...])
    o_ref[...] = _lrelu(m2 + sh, 0.01)


def _out_call(p, nf2, r2, f, interpret=False):
    dout2 = p['mlp2']['W'].shape[0]
    n = f.shape[1]
    rs = lambda a: a.reshape(-1, 1)
    return pl.pallas_call(
        _out_kernel,
        out_shape=jax.ShapeDtypeStruct((dout2, n), jnp.float32),
        interpret=interpret,
    )(nf2, r2, f,
      p['pool2_score']['W'], rs(p['pool2_score']['b']),
      p['pool2_mlp']['W'], rs(p['pool2_mlp']['b']),
      rs(p['pool2_mlp']['gamma']), rs(p['pool2_mlp']['beta']),
      p['mlp2']['W'], rs(p['mlp2']['b']),
      rs(p['mlp2']['gamma']), rs(p['mlp2']['beta']),
      p['shortcut']['W'], rs(p['shortcut']['b']),
      rs(p['shortcut']['gamma']), rs(p['shortcut']['beta']))


def _gather_nb(table_cn, idx4n):
    g = jnp.take(table_cn, idx4n.reshape(-1), axis=1)
    return g.reshape(table_cn.shape[0], 4, -1).transpose(1, 0, 2)


def _lfa(p, coords, ckT, f, interpret=False):
    idx8 = _knn(coords, ckT, 4, interpret=interpret)
    idx4n = idx8[:, :4].T
    x1 = _smlp_call(p['mlp1'], f, 0.2, interpret=interpret)
    nbc = _gather_nb(ckT, idx4n)
    nf1 = _gather_nb(x1, idx4n)
    x2, r2 = _mid_call(p, ckT, nbc, nf1, interpret=interpret)
    nf2 = _gather_nb(x2, idx4n)
    return _out_call(p, nf2, r2, f, interpret=interpret)


def _run(x, params, interpret=False):
    N = x.shape[1]
    xT = x[0].T
    p0 = {'W': params['fc0']['W'], 'b': params['fc0']['b'],
          'gamma': params['bn0']['gamma'], 'beta': params['bn0']['beta']}
    h = _smlp_call(p0, xT, 0.2, interpret=interpret)
    ratio = 1
    for name in ('lfa0', 'lfa1', 'lfa2', 'lfa3'):
        n = N // ratio
        coords = x[0, :n, :3]
        h = _lfa(params[name], coords, xT[:, :n], h, interpret=interpret)
        ratio *= 4
        h = h[:, :N // ratio]
    out = _smlp_call(params['mlp'], h, 0.2, interpret=interpret)
    return out[None, :, :, None]


def kernel(x, params):
    return _run(x, params)
